# baseline (speedup 1.0000x reference)
import functools

import jax
import jax.numpy as jnp
import numpy as np
from jax import lax
from jax.experimental import pallas as pl
from jax.experimental.pallas import tpu as pltpu
from jax.experimental.pallas import tpu_sc as plsc

N = 4096
CTX = 4096
ACT = CTX // 2
P_CHUNKS = 64
D = 128

EMB_BLK = 512
ARG_BLK = 256

MAXSIZE = 9223372036854775807.0
INT_MAX = 2147483647


def _pooling_matrices():
    l = np.arange(CTX)
    pa = (l[:, None] // (CTX // P_CHUNKS) == np.arange(P_CHUNKS)[None, :])
    pa = pa.astype(np.float32) / (CTX // P_CHUNKS)
    la = np.arange(ACT)
    pb = (la[:, None] // (ACT // P_CHUNKS) == np.arange(P_CHUNKS)[None, :])
    pb = pb.astype(np.float32) / (ACT // P_CHUNKS)
    return pa, pb


_P_AP, _P_ACTV = _pooling_matrices()


def _fused_body(x_ref, pap_ref, pactv_ref, w_ref, b_ref, hrow_ref, hcol_ref,
                eap_ref, eactv_ref, idx_ref, ea_scr):
    i = pl.program_id(0)
    nblk = N // EMB_BLK

    @pl.when(i < nblk)
    def _embed_phase():
        x = x_ref[...]
        w = w_ref[...]
        b = b_ref[...]
        pooled_ap = jnp.dot(x, pap_ref[...],
                            preferred_element_type=jnp.float32)
        pooled_actv = jnp.dot(x[:, :ACT], pactv_ref[...],
                              precision=lax.Precision.HIGHEST,
                              preferred_element_type=jnp.float32)
        eap_ref[...] = jnp.tanh(
            jnp.dot(pooled_ap, w, preferred_element_type=jnp.float32) + b)
        eactv = jnp.tanh(
            jnp.dot(pooled_actv, w, preferred_element_type=jnp.float32) + b)
        eactv_ref[...] = eactv
        ea_scr[pl.ds(i * EMB_BLK, EMB_BLK), :] = eactv

    @pl.when(i >= nblk)
    def _argmin_phase():
        j = i - nblk
        idx_ref[...] = jnp.zeros((1, 1, ARG_BLK), jnp.int32)
        return
        ef = ea_scr[...]
        er = ea_scr[pl.ds(j * ARG_BLK, ARG_BLK), :]
        hr = hrow_ref[...]
        hc = hcol_ref[...]
        g = lax.dot_general(er, ef, (((1,), (1,)), ((), ())),
                            preferred_element_type=jnp.float32)
        sqr = jnp.sum(er * er, axis=1)
        sqf = jnp.sum(ef * ef, axis=1)
        d2 = (sqr[:, None] + sqf[None, :]) - 2.0 * g
        d2 = jnp.maximum(d2, 0.0)
        same = hr == hc
        d2 = jnp.where(same, jnp.float32(MAXSIZE), d2)
        rmin = jnp.min(d2, axis=1)
        iota = lax.broadcasted_iota(jnp.int32, (ARG_BLK, N), 1)
        cand = jnp.where(d2 == rmin[:, None], iota, jnp.int32(INT_MAX))
        idx_ref[...] = jnp.min(cand, axis=1).reshape(1, 1, ARG_BLK)


def _sc_gather(table_hbm, idx_hbm, out_hbm, idx_v, rows_v, sem):
    wid = lax.axis_index("s") * 2 + lax.axis_index("c")
    b_per_w = N // 32
    base = wid * b_per_w
    pltpu.sync_copy(idx_hbm.at[pl.ds(base, b_per_w)], idx_v)
    pltpu.async_copy(table_hbm.at[idx_v], rows_v, sem).wait()
    pltpu.sync_copy(rows_v, out_hbm.at[pl.ds(base, b_per_w)])


def kernel(context, host, W, b):
    b2 = b.reshape(1, D)
    host_i32 = host.astype(jnp.int32)

    nblk = N // EMB_BLK
    e_ap, e_actv, idx3 = pl.pallas_call(
        _fused_body,
        grid=(nblk + N // ARG_BLK,),
        in_specs=[
            pl.BlockSpec((EMB_BLK, CTX), lambda i: (jnp.minimum(i, nblk - 1), 0)),
            pl.BlockSpec((CTX, P_CHUNKS), lambda i: (0, 0)),
            pl.BlockSpec((ACT, P_CHUNKS), lambda i: (0, 0)),
            pl.BlockSpec((P_CHUNKS, D), lambda i: (0, 0)),
            pl.BlockSpec((1, D), lambda i: (0, 0)),
            pl.BlockSpec((ARG_BLK, 1),
                         lambda i: (jnp.maximum(i - nblk, 0), 0)),
            pl.BlockSpec((1, N), lambda i: (0, 0)),
        ],
        out_specs=[
            pl.BlockSpec((EMB_BLK, D), lambda i: (jnp.minimum(i, nblk - 1), 0)),
            pl.BlockSpec((EMB_BLK, D), lambda i: (jnp.minimum(i, nblk - 1), 0)),
            pl.BlockSpec((1, 1, ARG_BLK),
                         lambda i: (jnp.maximum(i - nblk, 0), 0, 0)),
        ],
        out_shape=[
            jax.ShapeDtypeStruct((N, D), jnp.float32),
            jax.ShapeDtypeStruct((N, D), jnp.float32),
            jax.ShapeDtypeStruct((N // ARG_BLK, 1, ARG_BLK), jnp.int32),
        ],
        scratch_shapes=[pltpu.VMEM((N, D), jnp.float32)],
    )(context, jnp.asarray(_P_AP), jnp.asarray(_P_ACTV), W, b2,
      host_i32.reshape(N, 1), host_i32.reshape(1, N))
    idx = idx3.reshape(N)

    mesh = plsc.VectorSubcoreMesh(core_axis_name="c", subcore_axis_name="s",
                                  num_cores=2, num_subcores=16)
    b_per_w = N // 32
    e_an = pl.kernel(
        _sc_gather,
        out_type=jax.ShapeDtypeStruct((N, D), jnp.float32),
        mesh=mesh,
        scratch_types=[
            pltpu.VMEM((b_per_w,), jnp.int32),
            pltpu.VMEM((b_per_w, D), jnp.float32),
            pltpu.SemaphoreType.DMA,
        ],
    )(e_actv, idx)

    return (e_actv, e_ap, e_an)


# speedup vs baseline: 1.8346x; 1.8346x over previous
import functools

import jax
import jax.numpy as jnp
import numpy as np
from jax import lax
from jax.experimental import pallas as pl
from jax.experimental.pallas import tpu as pltpu
from jax.experimental.pallas import tpu_sc as plsc

N = 4096
CTX = 4096
ACT = CTX // 2
P_CHUNKS = 64
D = 128

EMB_BLK = 512
ARG_BLK = 512

MAXSIZE = 9223372036854775807.0
INT_MAX = 2147483647


def _pooling_matrices():
    l = np.arange(CTX)
    pa = (l[:, None] // (CTX // P_CHUNKS) == np.arange(P_CHUNKS)[None, :])
    pa = pa.astype(np.float32) / (CTX // P_CHUNKS)
    la = np.arange(ACT)
    pb = (la[:, None] // (ACT // P_CHUNKS) == np.arange(P_CHUNKS)[None, :])
    pb = pb.astype(np.float32) / (ACT // P_CHUNKS)
    return pa, pb


_P_AP, _P_ACTV = _pooling_matrices()


def _fused_body(x_ref, pap_ref, pactv_ref, w_ref, b_ref, hrow_ref, hcol_ref,
                eap_ref, eactv_ref, idx_ref, ea_scr):
    i = pl.program_id(0)
    nblk = N // EMB_BLK

    @pl.when(i < nblk)
    def _embed_phase():
        x = x_ref[...]
        w = w_ref[...]
        b = b_ref[...]
        pooled_ap = jnp.dot(x, pap_ref[...],
                            preferred_element_type=jnp.float32)
        pooled_actv = jnp.dot(x[:, :ACT], pactv_ref[...],
                              precision=lax.Precision.HIGHEST,
                              preferred_element_type=jnp.float32)
        eap_ref[...] = jnp.tanh(
            jnp.dot(pooled_ap, w, preferred_element_type=jnp.float32) + b)
        eactv = jnp.tanh(
            jnp.dot(pooled_actv, w, preferred_element_type=jnp.float32) + b)
        eactv_ref[...] = eactv
        ea_scr[pl.ds(i * EMB_BLK, EMB_BLK), :] = eactv

    @pl.when(i >= nblk)
    def _argmin_phase():
        j = i - nblk
        ef = ea_scr[...]
        er = ea_scr[pl.ds(j * ARG_BLK, ARG_BLK), :]
        hr = hrow_ref[...]
        hc = hcol_ref[...]
        g = lax.dot_general(er, ef, (((1,), (1,)), ((), ())),
                            preferred_element_type=jnp.float32)
        sqr = jnp.sum(er * er, axis=1)
        sqf = jnp.sum(ef * ef, axis=1)
        d2 = (sqr[:, None] + sqf[None, :]) - 2.0 * g
        d2 = jnp.maximum(d2, 0.0)
        same = hr == hc
        d2 = jnp.where(same, jnp.float32(MAXSIZE), d2)
        nchunk = N // 128
        d2r = d2.reshape(ARG_BLK, nchunk, 128)
        m1 = jnp.min(d2r, axis=1)
        i3 = lax.broadcasted_iota(jnp.int32, (ARG_BLK, nchunk, 128), 1)
        c1 = jnp.min(jnp.where(d2r == m1[:, None, :], i3, jnp.int32(INT_MAX)),
                     axis=1)
        rmin = jnp.min(m1, axis=1)
        lane = lax.broadcasted_iota(jnp.int32, (ARG_BLK, 128), 1)
        col = c1 * 128 + lane
        cand = jnp.where(m1 == rmin[:, None], col, jnp.int32(INT_MAX))
        idx_ref[...] = jnp.min(cand, axis=1).reshape(1, 1, ARG_BLK)


def _sc_gather(table_hbm, idx_hbm, out_hbm, idx_v, rows_v, sem):
    wid = lax.axis_index("s") * 2 + lax.axis_index("c")
    b_per_w = N // 32
    base = wid * b_per_w
    pltpu.sync_copy(idx_hbm.at[pl.ds(base, b_per_w)], idx_v)
    pltpu.async_copy(table_hbm.at[idx_v], rows_v, sem).wait()
    pltpu.sync_copy(rows_v, out_hbm.at[pl.ds(base, b_per_w)])


def kernel(context, host, W, b):
    b2 = b.reshape(1, D)
    host_i32 = host.astype(jnp.int32)

    nblk = N // EMB_BLK
    e_ap, e_actv, idx3 = pl.pallas_call(
        _fused_body,
        grid=(nblk + N // ARG_BLK,),
        in_specs=[
            pl.BlockSpec((EMB_BLK, CTX), lambda i: (jnp.minimum(i, nblk - 1), 0)),
            pl.BlockSpec((CTX, P_CHUNKS), lambda i: (0, 0)),
            pl.BlockSpec((ACT, P_CHUNKS), lambda i: (0, 0)),
            pl.BlockSpec((P_CHUNKS, D), lambda i: (0, 0)),
            pl.BlockSpec((1, D), lambda i: (0, 0)),
            pl.BlockSpec((ARG_BLK, 1),
                         lambda i: (jnp.maximum(i - nblk, 0), 0)),
            pl.BlockSpec((1, N), lambda i: (0, 0)),
        ],
        out_specs=[
            pl.BlockSpec((EMB_BLK, D), lambda i: (jnp.minimum(i, nblk - 1), 0)),
            pl.BlockSpec((EMB_BLK, D), lambda i: (jnp.minimum(i, nblk - 1), 0)),
            pl.BlockSpec((1, 1, ARG_BLK),
                         lambda i: (jnp.maximum(i - nblk, 0), 0, 0)),
        ],
        out_shape=[
            jax.ShapeDtypeStruct((N, D), jnp.float32),
            jax.ShapeDtypeStruct((N, D), jnp.float32),
            jax.ShapeDtypeStruct((N // ARG_BLK, 1, ARG_BLK), jnp.int32),
        ],
        scratch_shapes=[pltpu.VMEM((N, D), jnp.float32)],
    )(context, jnp.asarray(_P_AP), jnp.asarray(_P_ACTV), W, b2,
      host_i32.reshape(N, 1), host_i32.reshape(1, N))
    idx = idx3.reshape(N)

    mesh = plsc.VectorSubcoreMesh(core_axis_name="c", subcore_axis_name="s",
                                  num_cores=2, num_subcores=16)
    b_per_w = N // 32
    e_an = pl.kernel(
        _sc_gather,
        out_type=jax.ShapeDtypeStruct((N, D), jnp.float32),
        mesh=mesh,
        scratch_types=[
            pltpu.VMEM((b_per_w,), jnp.int32),
            pltpu.VMEM((b_per_w, D), jnp.float32),
            pltpu.SemaphoreType.DMA,
        ],
    )(e_actv, idx)

    return (e_actv, e_ap, e_an)


# speedup vs baseline: 2.2016x; 1.2001x over previous
import functools

import jax
import jax.numpy as jnp
import numpy as np
from jax import lax
from jax.experimental import pallas as pl
from jax.experimental.pallas import tpu as pltpu
from jax.experimental.pallas import tpu_sc as plsc

N = 4096
CTX = 4096
ACT = CTX // 2
P_CHUNKS = 64
D = 128

EMB_BLK = 512
ARG_BLK = 256

MAXSIZE = 9223372036854775807.0
INT_MAX = 2147483647


def _pooling_matrices():
    l = np.arange(CTX)
    pa = (l[:, None] // (CTX // P_CHUNKS) == np.arange(P_CHUNKS)[None, :])
    pa = pa.astype(np.float32) / (CTX // P_CHUNKS)
    la = np.arange(ACT)
    pb = (la[:, None] // (ACT // P_CHUNKS) == np.arange(P_CHUNKS)[None, :])
    pb = pb.astype(np.float32) / (ACT // P_CHUNKS)
    return pa, pb


_P_AP, _P_ACTV = _pooling_matrices()


def _fused_body(x_ref, pap_ref, pactv_ref, w_ref, b_ref, hrow_ref, hcol_ref,
                eap_ref, eactv_ref, idx_ref, ea_scr):
    i = pl.program_id(0)
    nblk = N // EMB_BLK

    @pl.when(i < nblk)
    def _embed_phase():
        x = x_ref[...]
        w = w_ref[...]
        b = b_ref[...]
        pooled_ap = jnp.dot(x, pap_ref[...],
                            preferred_element_type=jnp.float32)
        pooled_actv = jnp.dot(x[:, :ACT], pactv_ref[...],
                              precision=lax.Precision.HIGHEST,
                              preferred_element_type=jnp.float32)
        eap_ref[...] = jnp.tanh(
            jnp.dot(pooled_ap, w, preferred_element_type=jnp.float32) + b)
        eactv = jnp.tanh(
            jnp.dot(pooled_actv, w, preferred_element_type=jnp.float32) + b)
        eactv_ref[...] = eactv
        ea_scr[pl.ds(i * EMB_BLK, EMB_BLK), :] = eactv

    @pl.when(i >= nblk)
    def _argmin_phase():
        j = i - nblk
        ef = ea_scr[...]
        er = ea_scr[pl.ds(j * ARG_BLK, ARG_BLK), :]
        hr = hrow_ref[...]
        hc = hcol_ref[...]
        g = lax.dot_general(er, ef, (((1,), (1,)), ((), ())),
                            preferred_element_type=jnp.float32)
        sqr = jnp.sum(er * er, axis=1)
        sqf = jnp.sum(ef * ef, axis=1)
        d2 = (sqr[:, None] + sqf[None, :]) - 2.0 * g
        d2 = jnp.maximum(d2, 0.0)
        same = hr == hc
        d2 = jnp.where(same, jnp.float32(MAXSIZE), d2)
        idx_ref[...] = jnp.argmin(d2, axis=1).astype(jnp.int32).reshape(1, 1, ARG_BLK)


def _sc_gather(table_hbm, idx_hbm, out_hbm, idx_v, rows_v, sem):
    wid = lax.axis_index("s") * 2 + lax.axis_index("c")
    b_per_w = N // 32
    base = wid * b_per_w
    pltpu.sync_copy(idx_hbm.at[pl.ds(base, b_per_w)], idx_v)
    pltpu.async_copy(table_hbm.at[idx_v], rows_v, sem).wait()
    pltpu.sync_copy(rows_v, out_hbm.at[pl.ds(base, b_per_w)])


def kernel(context, host, W, b):
    b2 = b.reshape(1, D)
    host_i32 = host.astype(jnp.int32)

    nblk = N // EMB_BLK
    e_ap, e_actv, idx3 = pl.pallas_call(
        _fused_body,
        grid=(nblk + N // ARG_BLK,),
        in_specs=[
            pl.BlockSpec((EMB_BLK, CTX), lambda i: (jnp.minimum(i, nblk - 1), 0)),
            pl.BlockSpec((CTX, P_CHUNKS), lambda i: (0, 0)),
            pl.BlockSpec((ACT, P_CHUNKS), lambda i: (0, 0)),
            pl.BlockSpec((P_CHUNKS, D), lambda i: (0, 0)),
            pl.BlockSpec((1, D), lambda i: (0, 0)),
            pl.BlockSpec((ARG_BLK, 1),
                         lambda i: (jnp.maximum(i - nblk, 0), 0)),
            pl.BlockSpec((1, N), lambda i: (0, 0)),
        ],
        out_specs=[
            pl.BlockSpec((EMB_BLK, D), lambda i: (jnp.minimum(i, nblk - 1), 0)),
            pl.BlockSpec((EMB_BLK, D), lambda i: (jnp.minimum(i, nblk - 1), 0)),
            pl.BlockSpec((1, 1, ARG_BLK),
                         lambda i: (jnp.maximum(i - nblk, 0), 0, 0)),
        ],
        out_shape=[
            jax.ShapeDtypeStruct((N, D), jnp.float32),
            jax.ShapeDtypeStruct((N, D), jnp.float32),
            jax.ShapeDtypeStruct((N // ARG_BLK, 1, ARG_BLK), jnp.int32),
        ],
        scratch_shapes=[pltpu.VMEM((N, D), jnp.float32)],
    )(context, jnp.asarray(_P_AP), jnp.asarray(_P_ACTV), W, b2,
      host_i32.reshape(N, 1), host_i32.reshape(1, N))
    idx = idx3.reshape(N)

    mesh = plsc.VectorSubcoreMesh(core_axis_name="c", subcore_axis_name="s",
                                  num_cores=2, num_subcores=16)
    b_per_w = N // 32
    e_an = pl.kernel(
        _sc_gather,
        out_type=jax.ShapeDtypeStruct((N, D), jnp.float32),
        mesh=mesh,
        scratch_types=[
            pltpu.VMEM((b_per_w,), jnp.int32),
            pltpu.VMEM((b_per_w, D), jnp.float32),
            pltpu.SemaphoreType.DMA,
        ],
    )(e_actv, idx)

    return (e_actv, e_ap, e_an)


# speedup vs baseline: 2.3417x; 1.0636x over previous
import functools

import jax
import jax.numpy as jnp
import numpy as np
from jax import lax
from jax.experimental import pallas as pl
from jax.experimental.pallas import tpu as pltpu
from jax.experimental.pallas import tpu_sc as plsc

N = 4096
CTX = 4096
ACT = CTX // 2
P_CHUNKS = 64
D = 128

EMB_BLK = 1024
ARG_BLK = 512

MAXSIZE = 9223372036854775807.0
INT_MAX = 2147483647


def _pooling_matrices():
    l = np.arange(CTX)
    pa = (l[:, None] // (CTX // P_CHUNKS) == np.arange(P_CHUNKS)[None, :])
    pa = pa.astype(np.float32) / (CTX // P_CHUNKS)
    la = np.arange(ACT)
    pb = (la[:, None] // (ACT // P_CHUNKS) == np.arange(P_CHUNKS)[None, :])
    pb = pb.astype(np.float32) / (ACT // P_CHUNKS)
    return pa, pb


_P_AP, _P_ACTV = _pooling_matrices()


def _fused_body(x_ref, pap_ref, pactv_ref, w_ref, b_ref, hrow_ref, hcol_ref,
                eap_ref, eactv_ref, idx_ref, ea_scr):
    i = pl.program_id(0)
    nblk = N // EMB_BLK

    @pl.when(i < nblk)
    def _embed_phase():
        x = x_ref[...]
        w = w_ref[...]
        b = b_ref[...]
        pooled_ap = jnp.dot(x, pap_ref[...],
                            preferred_element_type=jnp.float32)
        pooled_actv = jnp.dot(x[:, :ACT], pactv_ref[...],
                              precision=lax.Precision.HIGHEST,
                              preferred_element_type=jnp.float32)
        eap_ref[...] = jnp.tanh(
            jnp.dot(pooled_ap, w, preferred_element_type=jnp.float32) + b)
        eactv = jnp.tanh(
            jnp.dot(pooled_actv, w, preferred_element_type=jnp.float32) + b)
        eactv_ref[...] = eactv
        ea_scr[pl.ds(i * EMB_BLK, EMB_BLK), :] = eactv

    @pl.when(i >= nblk)
    def _argmin_phase():
        j = i - nblk
        ef = ea_scr[...]
        er = ea_scr[pl.ds(j * ARG_BLK, ARG_BLK), :]
        hr = hrow_ref[...]
        hc = hcol_ref[...]
        g = lax.dot_general(er, ef, (((1,), (1,)), ((), ())),
                            preferred_element_type=jnp.float32)
        sqr = jnp.sum(er * er, axis=1)
        sqf = jnp.sum(ef * ef, axis=1)
        d2 = (sqr[:, None] + sqf[None, :]) - 2.0 * g
        d2 = jnp.maximum(d2, 0.0)
        same = hr == hc
        d2 = jnp.where(same, jnp.float32(MAXSIZE), d2)
        idx_ref[...] = jnp.argmin(d2, axis=1).astype(jnp.int32).reshape(1, 1, ARG_BLK)


def _sc_gather(table_hbm, idx_hbm, out_hbm, idx_v, rows_v, sem):
    wid = lax.axis_index("s") * 2 + lax.axis_index("c")
    b_per_w = N // 32
    base = wid * b_per_w
    pltpu.sync_copy(idx_hbm.at[pl.ds(base, b_per_w)], idx_v)
    pltpu.async_copy(table_hbm.at[idx_v], rows_v, sem).wait()
    pltpu.sync_copy(rows_v, out_hbm.at[pl.ds(base, b_per_w)])


def kernel(context, host, W, b):
    b2 = b.reshape(1, D)
    host_i32 = host.astype(jnp.int32)

    nblk = N // EMB_BLK
    e_ap, e_actv, idx3 = pl.pallas_call(
        _fused_body,
        grid=(nblk + N // ARG_BLK,),
        in_specs=[
            pl.BlockSpec((EMB_BLK, CTX), lambda i: (jnp.minimum(i, nblk - 1), 0)),
            pl.BlockSpec((CTX, P_CHUNKS), lambda i: (0, 0)),
            pl.BlockSpec((ACT, P_CHUNKS), lambda i: (0, 0)),
            pl.BlockSpec((P_CHUNKS, D), lambda i: (0, 0)),
            pl.BlockSpec((1, D), lambda i: (0, 0)),
            pl.BlockSpec((ARG_BLK, 1),
                         lambda i: (jnp.maximum(i - nblk, 0), 0)),
            pl.BlockSpec((1, N), lambda i: (0, 0)),
        ],
        out_specs=[
            pl.BlockSpec((EMB_BLK, D), lambda i: (jnp.minimum(i, nblk - 1), 0)),
            pl.BlockSpec((EMB_BLK, D), lambda i: (jnp.minimum(i, nblk - 1), 0)),
            pl.BlockSpec((1, 1, ARG_BLK),
                         lambda i: (jnp.maximum(i - nblk, 0), 0, 0)),
        ],
        out_shape=[
            jax.ShapeDtypeStruct((N, D), jnp.float32),
            jax.ShapeDtypeStruct((N, D), jnp.float32),
            jax.ShapeDtypeStruct((N // ARG_BLK, 1, ARG_BLK), jnp.int32),
        ],
        scratch_shapes=[pltpu.VMEM((N, D), jnp.float32)],
    )(context, jnp.asarray(_P_AP), jnp.asarray(_P_ACTV), W, b2,
      host_i32.reshape(N, 1), host_i32.reshape(1, N))
    idx = idx3.reshape(N)

    mesh = plsc.VectorSubcoreMesh(core_axis_name="c", subcore_axis_name="s",
                                  num_cores=2, num_subcores=16)
    b_per_w = N // 32
    e_an = pl.kernel(
        _sc_gather,
        out_type=jax.ShapeDtypeStruct((N, D), jnp.float32),
        mesh=mesh,
        scratch_types=[
            pltpu.VMEM((b_per_w,), jnp.int32),
            pltpu.VMEM((b_per_w, D), jnp.float32),
            pltpu.SemaphoreType.DMA,
        ],
    )(e_actv, idx)

    return (e_actv, e_ap, e_an)
